# baseline (speedup 1.0000x reference)
import functools

import jax
import jax.numpy as jnp
from jax import lax
from jax.experimental import pallas as pl
from jax.experimental.pallas import tpu as pltpu
from jax.experimental.pallas import tpu_sc as plsc


@functools.cache
def _detile_idx_kernel(B: int, L: int):
    info = plsc.get_sparse_core_info()
    NC, NS = info.num_cores, info.num_subcores
    NW = NC * NS
    NB = B // 128
    assert NB % NW == 0
    BPW = NB // NW
    C8 = (L + 7) // 8 * 8
    full = L // 8
    rem = L - full * 8
    n_tiles = C8 // 8

    mesh = plsc.VectorSubcoreMesh(core_axis_name="c", subcore_axis_name="s")

    @functools.partial(
        pl.kernel,
        mesh=mesh,
        out_type=jax.ShapeDtypeStruct((NB * C8 * 128,), jnp.int32),
        scratch_types=[
            pltpu.VMEM((C8, 128), jnp.int32),
            pltpu.VMEM((C8 * 128,), jnp.int32),
            pltpu.SemaphoreType.DMA,
            pltpu.SemaphoreType.DMA,
        ],
    )
    def k(idxt_hbm, out_hbm, st_v, f_v, sem_in, sem_out):
        wid = lax.axis_index("s") * NC + lax.axis_index("c")
        for blk in range(BPW):
            t = wid * BPW + blk
            cps = []
            for a in range(n_tiles):
                h = 8 if (a < full) else rem
                cps.append(pltpu.async_copy(
                    idxt_hbm.at[pl.ds(a * 8, h), pl.ds(t * 128, 128)],
                    st_v.at[pl.ds(a * 8, h), :], sem_in))
            for cp in cps:
                cp.wait()
            for j in range(L):
                for kk in range(8):
                    f_v[pl.ds(j * 128 + 16 * kk, 16)] = st_v[j, pl.ds(16 * kk, 16)]
            pltpu.async_copy(
                f_v, out_hbm.at[pl.ds(t * C8 * 128, C8 * 128)], sem_out).wait()

    return k


@functools.cache
def _pooled_sum_kernel(B: int, L: int, V: int, D: int, CB: int):
    info = plsc.get_sparse_core_info()
    NC, NS = info.num_cores, info.num_subcores
    NW = NC * NS
    NB = B // 128
    assert NB % NW == 0
    BPW = NB // NW
    C8 = (L + 7) // 8 * 8
    CPB = 128 // CB
    assert CPB * CB == 128 and CB % 16 == 0
    assert L % 2 == 0

    mesh = plsc.VectorSubcoreMesh(core_axis_name="c", subcore_axis_name="s")

    @functools.partial(
        pl.kernel,
        mesh=mesh,
        out_type=jax.ShapeDtypeStruct((B, D), jnp.float32),
        compiler_params=pltpu.CompilerParams(use_tc_tiling_on_sc=False),
        scratch_types=[
            pltpu.VMEM((C8 * 128,), jnp.int32),
            pltpu.VMEM((CB * L,), jnp.int32),
            pltpu.VMEM((CB * L,), jnp.int32),
            pltpu.VMEM((CB * L, D), jnp.float32),
            pltpu.VMEM((CB * L, D), jnp.float32),
            pltpu.VMEM((CB, D), jnp.float32),
            pltpu.VMEM((CB, D), jnp.float32),
            pltpu.SemaphoreType.DMA,
            pltpu.SemaphoreType.DMA,
        ],
    )
    def k(table_hbm, idx_hbm, out_hbm, st_v, idx0, idx1, rows0, rows1,
          acc0, acc1, sem0, sem1):
        wid = lax.axis_index("s") * NC + lax.axis_index("c")

        def repack_and_fire(cc, idx_v, rows_v, sem):
            col = cc * CB
            for j in range(L):
                for kk in range(CB // 16):
                    idx_v[pl.ds(j * CB + 16 * kk, 16)] = (
                        st_v[pl.ds(j * 128 + col + 16 * kk, 16)])
            pltpu.async_copy(table_hbm.at[idx_v], rows_v, sem)

        def accum_store(t, cc, rows_v, acc_v):
            def batch_body(b, carry2):
                z = jnp.zeros((16,), jnp.float32)
                a0 = a1 = a2 = a3 = z
                for j in range(0, L, 2):
                    a0 = a0 + rows_v[j * CB + b, pl.ds(0, 16)]
                    a1 = a1 + rows_v[j * CB + b, pl.ds(16, 16)]
                    a2 = a2 + rows_v[(j + 1) * CB + b, pl.ds(0, 16)]
                    a3 = a3 + rows_v[(j + 1) * CB + b, pl.ds(16, 16)]
                acc_v[b, pl.ds(0, 16)] = a0 + a2
                acc_v[b, pl.ds(16, 16)] = a1 + a3
                return carry2

            lax.fori_loop(0, CB, batch_body, 0)
            pltpu.sync_copy(acc_v, out_hbm.at[pl.ds(t * 128 + cc * CB, CB)])

        def wait_gather(idx_v, rows_v, sem):
            pltpu.make_async_copy(table_hbm.at[idx_v], rows_v, sem).wait()

        bufs = [(idx0, rows0, acc0, sem0), (idx1, rows1, acc1, sem1)]

        def block_body(blk, carry):
            t = wid * BPW + blk
            pltpu.sync_copy(idx_hbm.at[pl.ds(t * C8 * 128, C8 * 128)], st_v)
            repack_and_fire(0, *bufs[0][:2], bufs[0][3])
            if CPB > 1:
                repack_and_fire(1, *bufs[1][:2], bufs[1][3])
            for cc in range(CPB):
                ib, rb, ab, sb = bufs[cc % 2]
                wait_gather(ib, rb, sb)
                accum_store(t, cc, rb, ab)
                if cc + 2 < CPB:
                    repack_and_fire(cc + 2, ib, rb, sb)
            return carry

        lax.fori_loop(0, BPW, block_body, 0)

    return k


_KT = 12


@functools.cache
def _table_detile_kernel(V: int, D: int):
    assert D % 8 == 0
    info = plsc.get_sparse_core_info()
    NC, NS = info.num_cores, info.num_subcores
    NW = NC * NS
    NT = (V + 127) // 128
    NTF = V // 128
    VR = V - NTF * 128
    NSC = NTF // _KT
    REM = NTF - NSC * _KT
    NQ = D // 8
    UNITS = NQ * NSC
    SW = _KT * 1024
    npairs = (UNITS // NW + 2) // 2

    mesh = plsc.VectorSubcoreMesh(core_axis_name="c", subcore_axis_name="s")

    @functools.partial(
        pl.kernel,
        mesh=mesh,
        out_type=jax.ShapeDtypeStruct((NQ * NT * 1024,), jnp.float32),
        scratch_types=[
            pltpu.VMEM((8, _KT * 128), jnp.float32),
            pltpu.VMEM((8, _KT * 128), jnp.float32),
            pltpu.VMEM((SW,), jnp.float32),
            pltpu.VMEM((SW,), jnp.float32),
            pltpu.SemaphoreType.DMA,
            pltpu.SemaphoreType.DMA,
            pltpu.SemaphoreType.DMA,
            pltpu.SemaphoreType.DMA,
        ],
    )
    def k(tt_hbm, out_hbm, st0, st1, f0, f1, si0, si1, so0, so1):
        wid = lax.axis_index("s") * NC + lax.axis_index("c")

        def fire_stage(u, st_v, sem):
            q = u // NSC
            t0 = (u % NSC) * _KT
            pltpu.async_copy(
                tt_hbm.at[pl.ds(q * 8, 8), pl.ds(t0 * 128, _KT * 128)],
                st_v, sem)

        def flatten(st_v, f_v):
            for m in range(_KT):
                for r in range(8):
                    for kk in range(8):
                        f_v[pl.ds(m * 1024 + r * 128 + 16 * kk, 16)] = (
                            st_v[r, pl.ds(m * 128 + 16 * kk, 16)])

        def process(u, p, st_v, f_v, s_in, s_out):
            pltpu.make_async_copy(
                tt_hbm.at[pl.ds(0, 8), pl.ds(0, _KT * 128)], st_v, s_in).wait()
            pos = ((u // NSC) * NT + (u % NSC) * _KT) * 1024

            @pl.when(p >= 1)
            def _():
                pltpu.make_async_copy(
                    f_v, out_hbm.at[pl.ds(pos, SW)], s_out).wait()

            flatten(st_v, f_v)
            pltpu.async_copy(f_v, out_hbm.at[pl.ds(pos, SW)], s_out)

        def pair_body(p, carry):
            ua = wid + (2 * p) * NW
            ub = wid + (2 * p + 1) * NW
            un = wid + (2 * p + 2) * NW

            @pl.when(ub < UNITS)
            def _():
                fire_stage(ub, st1, si1)

            @pl.when(ua < UNITS)
            def _():
                process(ua, p, st0, f0, si0, so0)

            @pl.when(un < UNITS)
            def _():
                fire_stage(un, st0, si0)

            @pl.when(ub < UNITS)
            def _():
                process(ub, p, st1, f1, si1, so1)

            return carry

        fire_stage(wid, st0, si0)
        lax.fori_loop(0, npairs, pair_body, 0)
        pltpu.make_async_copy(f0, out_hbm.at[pl.ds(0, SW)], so0).wait()
        pltpu.make_async_copy(f1, out_hbm.at[pl.ds(0, SW)], so1).wait()

        for q in range(NQ):
            for j in range(REM):
                u2 = q * REM + j

                @pl.when(wid == u2)
                def _(q=q, j=j):
                    t = NSC * _KT + j
                    pltpu.sync_copy(
                        tt_hbm.at[pl.ds(q * 8, 8), pl.ds(t * 128, 128)],
                        st0.at[:, pl.ds(0, 128)])
                    for r in range(8):
                        for kk in range(8):
                            f0[pl.ds(r * 128 + 16 * kk, 16)] = (
                                st0[r, pl.ds(16 * kk, 16)])
                    pltpu.sync_copy(
                        f0.at[pl.ds(0, 1024)],
                        out_hbm.at[pl.ds((q * NT + t) * 1024, 1024)])

    return k


@functools.cache
def _table_rm_kernel(V: int, D: int):
    assert D == 32
    info = plsc.get_sparse_core_info()
    NC, NS = info.num_cores, info.num_subcores
    NW = NC * NS
    NT = (V + 127) // 128
    NG = V // 128
    VR = V - NG * 128
    GW = 128 * D
    NQ = D // 8
    npairs = (NG // NW + 2) // 2

    mesh = plsc.VectorSubcoreMesh(core_axis_name="c", subcore_axis_name="s")

    @functools.partial(
        pl.kernel,
        mesh=mesh,
        out_type=jax.ShapeDtypeStruct((V * D,), jnp.float32),
        compiler_params=pltpu.CompilerParams(use_tc_tiling_on_sc=False,
                                             needs_layout_passes=False),
        scratch_types=[
            pltpu.VMEM((D, 128), jnp.float32),
            pltpu.VMEM((D, 128), jnp.float32),
            pltpu.VMEM((GW,), jnp.float32),
            pltpu.VMEM((GW,), jnp.float32),
            pltpu.SemaphoreType.DMA,
            pltpu.SemaphoreType.DMA,
            pltpu.SemaphoreType.DMA,
            pltpu.SemaphoreType.DMA,
        ],
    )
    def k(tiles_hbm, tail_hbm, out_hbm, st0, st1, f0, f1, si0, si1, so0, so1):
        wid = lax.axis_index("s") * NC + lax.axis_index("c")
        iota = lax.iota(jnp.int32, 16)

        def fire_stage(g, st_v, sem):
            for q in range(NQ):
                pltpu.async_copy(
                    tiles_hbm.at[pl.ds((q * NT + g) * 8, 8), :],
                    st_v.at[pl.ds(q * 8, 8), :], sem)

        def repack(st_v, f_v, w):
            for vl in range(w):
                col = jnp.full((16,), vl, jnp.int32)
                v0 = plsc.load_gather(st_v, [iota, col])
                v1 = plsc.load_gather(st_v, [iota + 16, col])
                f_v[pl.ds(vl * D, 16)] = v0
                f_v[pl.ds(vl * D + 16, 16)] = v1

        def process(g, p, st_v, f_v, s_in, s_out):
            for q in range(NQ):
                pltpu.make_async_copy(
                    tiles_hbm.at[pl.ds(0, 8), :],
                    st_v.at[pl.ds(q * 8, 8), :], s_in).wait()

            @pl.when(p >= 1)
            def _():
                pltpu.make_async_copy(
                    f_v, out_hbm.at[pl.ds(g * GW, GW)], s_out).wait()

            repack(st_v, f_v, 128)
            pltpu.async_copy(f_v, out_hbm.at[pl.ds(g * GW, GW)], s_out)

        def pair_body(p, carry):
            ga = wid + (2 * p) * NW
            gb = wid + (2 * p + 1) * NW
            gn = wid + (2 * p + 2) * NW

            @pl.when(gb < NG)
            def _():
                fire_stage(gb, st1, si1)

            @pl.when(ga < NG)
            def _():
                process(ga, p, st0, f0, si0, so0)

            @pl.when(gn < NG)
            def _():
                fire_stage(gn, st0, si0)

            @pl.when(gb < NG)
            def _():
                process(gb, p, st1, f1, si1, so1)

            return carry

        fire_stage(wid, st0, si0)
        lax.fori_loop(0, npairs, pair_body, 0)
        pltpu.make_async_copy(f0, out_hbm.at[pl.ds(0, GW)], so0).wait()
        pltpu.make_async_copy(f1, out_hbm.at[pl.ds(0, GW)], so1).wait()

        if VR:
            @pl.when(wid == NW - 1)
            def _():
                pltpu.sync_copy(tail_hbm,
                                out_hbm.at[pl.ds(NG * GW, VR * D)])

    return k


def _head_body(s1_ref, s2_ref, nl_ref, wl_ref, w1_ref, b1_ref, w2_ref,
               b2_ref, o_ref):
    x1 = s1_ref[...] / nl_ref[...]
    x2 = s2_ref[...] / wl_ref[...]
    h = jnp.tanh(jnp.concatenate([x1, x2], axis=1))
    u = lax.dot_general(h, w1_ref[...], (((1,), (1,)), ((), ())),
                        preferred_element_type=jnp.float32) + b1_ref[...]
    o_ref[...] = lax.dot_general(w2_ref[...], u, (((1,), (1,)), ((), ())),
                                 preferred_element_type=jnp.float32) + b2_ref[...]


def _head(s1, s2, ngram_len, word_len, W1, b1, W2, b2):
    B, D = s1.shape
    OUTV, OUTD = W2.shape
    BM = 512
    grid = (B // BM,)
    nl = ngram_len.reshape(B, 1)
    wl = word_len.reshape(B, 1)
    yt = pl.pallas_call(
        _head_body,
        grid=grid,
        in_specs=[
            pl.BlockSpec((BM, D), lambda i: (i, 0)),
            pl.BlockSpec((BM, D), lambda i: (i, 0)),
            pl.BlockSpec((BM, 1), lambda i: (i, 0)),
            pl.BlockSpec((BM, 1), lambda i: (i, 0)),
            pl.BlockSpec((OUTD, 2 * D), lambda i: (0, 0)),
            pl.BlockSpec((1, OUTD), lambda i: (0, 0)),
            pl.BlockSpec((OUTV, OUTD), lambda i: (0, 0)),
            pl.BlockSpec((OUTV, 1), lambda i: (0, 0)),
        ],
        out_specs=pl.BlockSpec((OUTV, BM), lambda i: (0, i)),
        out_shape=jax.ShapeDtypeStruct((OUTV, B), jnp.float32),
    )(s1, s2, nl, wl, W1, b1.reshape(1, OUTD), W2, b2.reshape(OUTV, 1))
    return yt.T


def kernel(words, word_len, ngrams, ngram_len, ngram_table, word_table,
           W1, b1, W2, b2):
    B, LW = words.shape
    _, LN = ngrams.shape
    WV, WD = word_table.shape
    NV, ND = ngram_table.shape

    ngrams_t = ngrams.astype(jnp.int32).T
    words_t = words.astype(jnp.int32).T

    ngflat = _detile_idx_kernel(B, LN)(ngrams_t)
    wdflat = _detile_idx_kernel(B, LW)(words_t)
    def relayout_table(table):
        V, D = table.shape
        NQ, NT = D // 8, (V + 127) // 128
        VR = V - (V // 128) * 128
        tail = (table[V - VR:, :].reshape(-1) if VR
                else jnp.zeros((1,), jnp.float32))
        tiles = _table_detile_kernel(V, D)(table.T).reshape(NQ * NT * 8, 128)
        return _table_rm_kernel(V, D)(tiles, tail).reshape(V, D)

    ngt = relayout_table(ngram_table)
    wdt = relayout_table(word_table)
    s1 = _pooled_sum_kernel(B, LN, NV, ND, 32)(ngt, ngflat)
    s2 = _pooled_sum_kernel(B, LW, WV, WD, 64)(wdt, wdflat)
    return _head(s1, s2, ngram_len, word_len, W1, b1, W2, b2)


# speedup vs baseline: 1.8469x; 1.8469x over previous
import functools

import jax
import jax.numpy as jnp
from jax import lax
from jax.experimental import pallas as pl
from jax.experimental.pallas import tpu as pltpu
from jax.experimental.pallas import tpu_sc as plsc


@functools.cache
def _pooled_sum_kernel(B: int, L: int, V: int, D: int, CB: int):
    info = plsc.get_sparse_core_info()
    NC, NS = info.num_cores, info.num_subcores
    NW = NC * NS
    assert B % (NW * CB) == 0 and CB % 16 == 0
    PB = B // NW
    n_chunks = PB // CB
    assert n_chunks % 2 == 0
    npairs = n_chunks // 2
    assert L % 2 == 0

    mesh = plsc.VectorSubcoreMesh(core_axis_name="c", subcore_axis_name="s")

    @functools.partial(
        pl.kernel,
        mesh=mesh,
        out_type=jax.ShapeDtypeStruct((B, D), jnp.float32),
        compiler_params=pltpu.CompilerParams(use_tc_tiling_on_sc=False),
        scratch_types=[
            pltpu.VMEM((L, CB), jnp.int32),
            pltpu.VMEM((L, CB), jnp.int32),
            pltpu.VMEM((CB * L,), jnp.int32),
            pltpu.VMEM((CB * L,), jnp.int32),
            pltpu.VMEM((CB * L, D), jnp.float32),
            pltpu.VMEM((CB * L, D), jnp.float32),
            pltpu.VMEM((CB, D), jnp.float32),
            pltpu.VMEM((CB, D), jnp.float32),
            pltpu.SemaphoreType.DMA,
            pltpu.SemaphoreType.DMA,
        ],
    )
    def k(table_hbm, idxt_hbm, out_hbm, st0, st1, idx0, idx1, rows0, rows1,
          acc0, acc1, sem0, sem1):
        wid = lax.axis_index("s") * NC + lax.axis_index("c")
        wbase = wid * PB

        def accum_chunk(rows_v, acc_v):
            def batch_body(b, carry2):
                z = jnp.zeros((16,), jnp.float32)
                a0 = a1 = a2 = a3 = z
                for j in range(0, L, 2):
                    a0 = a0 + rows_v[j * CB + b, pl.ds(0, 16)]
                    a1 = a1 + rows_v[j * CB + b, pl.ds(16, 16)]
                    a2 = a2 + rows_v[(j + 1) * CB + b, pl.ds(0, 16)]
                    a3 = a3 + rows_v[(j + 1) * CB + b, pl.ds(16, 16)]
                acc_v[b, pl.ds(0, 16)] = a0 + a2
                acc_v[b, pl.ds(16, 16)] = a1 + a3
                return carry2

            lax.fori_loop(0, CB, batch_body, 0)

        def stage_and_fire(c, st_v, idx_v, rows_v, sem):
            base = wbase + c * CB
            pltpu.sync_copy(idxt_hbm.at[:, pl.ds(base, CB)], st_v)
            for j in range(L):
                for kk in range(CB // 16):
                    idx_v[pl.ds(j * CB + 16 * kk, 16)] = st_v[j, pl.ds(16 * kk, 16)]
            pltpu.async_copy(table_hbm.at[idx_v], rows_v, sem)

        stage_and_fire(0, st0, idx0, rows0, sem0)

        def pair_body(i, carry):
            c0 = 2 * i
            stage_and_fire(c0 + 1, st1, idx1, rows1, sem1)
            pltpu.make_async_copy(table_hbm.at[idx0], rows0, sem0).wait()
            accum_chunk(rows0, acc0)
            pltpu.sync_copy(acc0, out_hbm.at[pl.ds(wbase + c0 * CB, CB)])

            @pl.when(i + 1 < npairs)
            def _():
                stage_and_fire(c0 + 2, st0, idx0, rows0, sem0)

            pltpu.make_async_copy(table_hbm.at[idx1], rows1, sem1).wait()
            accum_chunk(rows1, acc1)
            pltpu.sync_copy(acc1, out_hbm.at[pl.ds(wbase + (c0 + 1) * CB, CB)])
            return carry

        lax.fori_loop(0, npairs, pair_body, 0)

    return k


def _head_body(s1_ref, s2_ref, nl_ref, wl_ref, w1_ref, b1_ref, w2_ref,
               b2_ref, o_ref):
    x1 = s1_ref[...] / nl_ref[...]
    x2 = s2_ref[...] / wl_ref[...]
    h = jnp.tanh(jnp.concatenate([x1, x2], axis=1))
    u = lax.dot_general(h, w1_ref[...], (((1,), (1,)), ((), ())),
                        preferred_element_type=jnp.float32) + b1_ref[...]
    o_ref[...] = lax.dot_general(w2_ref[...], u, (((1,), (1,)), ((), ())),
                                 preferred_element_type=jnp.float32) + b2_ref[...]


def _head(s1, s2, ngram_len, word_len, W1, b1, W2, b2):
    B, D = s1.shape
    OUTV, OUTD = W2.shape
    BM = 512
    grid = (B // BM,)
    nl = ngram_len.reshape(B, 1)
    wl = word_len.reshape(B, 1)
    yt = pl.pallas_call(
        _head_body,
        grid=grid,
        in_specs=[
            pl.BlockSpec((BM, D), lambda i: (i, 0)),
            pl.BlockSpec((BM, D), lambda i: (i, 0)),
            pl.BlockSpec((BM, 1), lambda i: (i, 0)),
            pl.BlockSpec((BM, 1), lambda i: (i, 0)),
            pl.BlockSpec((OUTD, 2 * D), lambda i: (0, 0)),
            pl.BlockSpec((1, OUTD), lambda i: (0, 0)),
            pl.BlockSpec((OUTV, OUTD), lambda i: (0, 0)),
            pl.BlockSpec((OUTV, 1), lambda i: (0, 0)),
        ],
        out_specs=pl.BlockSpec((OUTV, BM), lambda i: (0, i)),
        out_shape=jax.ShapeDtypeStruct((OUTV, B), jnp.float32),
    )(s1, s2, nl, wl, W1, b1.reshape(1, OUTD), W2, b2.reshape(OUTV, 1))
    return yt.T


def kernel(words, word_len, ngrams, ngram_len, ngram_table, word_table,
           W1, b1, W2, b2):
    B, LW = words.shape
    _, LN = ngrams.shape
    WV, WD = word_table.shape
    NV, ND = ngram_table.shape

    ngrams_t = ngrams.astype(jnp.int32).T
    words_t = words.astype(jnp.int32).T

    s1 = _pooled_sum_kernel(B, LN, NV, ND, 32)(ngram_table, ngrams_t)
    s2 = _pooled_sum_kernel(B, LW, WV, WD, 64)(word_table, words_t)
    return _head(s1, s2, ngram_len, word_len, W1, b1, W2, b2)
